# Initial kernel scaffold; baseline (speedup 1.0000x reference)
import jax
import jax.numpy as jnp
from jax.experimental import pallas as pl


def kernel(x, edge_index, edge_attr, W1, b1, W2, b2, W3, b3, W4, b4):
    raise NotImplementedError("write your pallas kernel here")


# speedup vs baseline: 7.4981x; 7.4981x over previous
import functools

import jax
import jax.numpy as jnp
from jax import lax
from jax.experimental import pallas as pl
from jax.experimental.pallas import tpu as pltpu
from jax.experimental.pallas import tpu_sc as plsc

NN = 10000
EE = 320000
DD = 128
DH = DD // 2
NC = 2
NS = 16
NW = NC * NS
NPAD = 10240
RPT = NPAD // NS
EC = 128
EPT = EE // NW
NCHUNK = 80
NGCHUNK = NCHUNK + 2

_MESH = plsc.VectorSubcoreMesh(core_axis_name="c", subcore_axis_name="s")
_SC_PARAMS = pltpu.CompilerParams(use_tc_tiling_on_sc=False)


def _unpack_indices(pk_v, dst_v, nchunk):
    def _body(i, carry):
        r = i // 8
        sl = pl.ds((i % 8) * 16, 16)
        v = pk_v[r, sl]
        dst_v[r, sl] = lax.shift_right_logical(v, 14)
        pk_v[r, sl] = v & 0x3FFF
        return carry

    lax.fori_loop(0, nchunk * 8, _body, 0)


@functools.partial(
    pl.kernel,
    out_type=jax.ShapeDtypeStruct((NC, NPAD), jnp.float32),
    mesh=_MESH,
    compiler_params=_SC_PARAMS,
    scratch_types=[
        pltpu.VMEM((NCHUNK, EC), jnp.int32),
        pltpu.VMEM((NCHUNK, EC), jnp.int32),
        pltpu.VMEM((NCHUNK, EC), jnp.float32),
        pltpu.VMEM((RPT,), jnp.float32),
        pltpu.VMEM_SHARED((NPAD,), jnp.float32),
    ],
)
def _deg_kernel(pk_hbm, w_hbm, deg_hbm, pk_v, dst_v, w_v, zbuf, acc):
    c = lax.axis_index("c")
    s = lax.axis_index("s")
    wid = c * NS + s
    pltpu.sync_copy(pk_hbm.at[wid, pl.ds(0, NCHUNK)], pk_v)
    pltpu.sync_copy(w_hbm.at[wid], w_v)
    _unpack_indices(pk_v, dst_v, NCHUNK)

    def _zero(i, carry):
        zbuf[pl.ds(i * 16, 16)] = jnp.zeros((16,), jnp.float32)
        return carry

    lax.fori_loop(0, RPT // 16, _zero, 0)
    pltpu.sync_copy(zbuf, acc.at[pl.ds(s * RPT, RPT)])
    plsc.subcore_barrier()

    def _chunk(j, carry):
        pltpu.sync_copy(w_v.at[j], acc.at[dst_v.at[j]], add=True)
        return carry

    lax.fori_loop(0, NCHUNK, _chunk, 0)
    plsc.subcore_barrier()
    pltpu.sync_copy(acc.at[pl.ds(s * RPT, RPT)],
                    deg_hbm.at[c, pl.ds(s * RPT, RPT)])


@functools.partial(
    pl.kernel,
    out_type=jax.ShapeDtypeStruct((NC, 2, NPAD, DH), jnp.float32),
    mesh=_MESH,
    compiler_params=_SC_PARAMS,
    scratch_types=[
        pltpu.VMEM((NGCHUNK, EC), jnp.int32),
        pltpu.VMEM((NGCHUNK, EC), jnp.int32),
        pltpu.VMEM((NCHUNK, EC), jnp.float32),
        pltpu.VMEM((2, EC, DH), jnp.float32),
        pltpu.SemaphoreType.DMA,
        pltpu.SemaphoreType.DMA,
        pltpu.VMEM_SHARED((NPAD, DH), jnp.float32),
    ],
)
def _spmm_kernel(pk_hbm, w_hbm, ulo_hbm, uhi_hbm, out_hbm,
                 src_v, dst_v, w_v, rows_v, sem0, sem1, acc):
    c = lax.axis_index("c")
    s = lax.axis_index("s")
    wid = c * NS + s
    sems = (sem0, sem1)
    pltpu.sync_copy(pk_hbm.at[wid], src_v)
    pltpu.sync_copy(w_hbm.at[wid], w_v)
    _unpack_indices(src_v, dst_v, NGCHUNK)

    for h, u_hbm in enumerate((ulo_hbm, uhi_hbm)):
        def _zero(i, carry):
            for g in range(DH // 16):
                rows_v[0, i, pl.ds(g * 16, 16)] = jnp.zeros((16,), jnp.float32)
            return carry

        lax.fori_loop(0, EC, _zero, 0)
        for k in range(RPT // EC):
            pltpu.sync_copy(rows_v.at[0], acc.at[pl.ds(s * RPT + k * EC, EC)])
        plsc.subcore_barrier()

        for b in range(2):
            pltpu.async_copy(u_hbm.at[src_v.at[b]], rows_v.at[b], sems[b])

        def _do_chunk(jj, b):
            pltpu.make_async_copy(u_hbm.at[src_v.at[jj]], rows_v.at[b],
                                  sems[b]).wait()

            def _scale(g16, carry):
                w16 = w_v[jj, pl.ds(g16 * 16, 16)]
                for l in range(16):
                    e = g16 * 16 + l
                    wl = w16[l]
                    for g in range(DH // 16):
                        sl = pl.ds(g * 16, 16)
                        rows_v[b, e, sl] = rows_v[b, e, sl] * wl
                return carry

            lax.fori_loop(0, EC // 16, _scale, 0)
            pltpu.sync_copy(rows_v.at[b], acc.at[dst_v.at[jj]], add=True)
            pltpu.async_copy(u_hbm.at[src_v.at[jj + 2]], rows_v.at[b], sems[b])

        def _outer(i, carry):
            for b in range(2):
                _do_chunk(i * 2 + b, b)
            return carry

        lax.fori_loop(0, NCHUNK // 2, _outer, 0)
        for b in range(2):
            pltpu.make_async_copy(u_hbm.at[src_v.at[NCHUNK + b]],
                                  rows_v.at[b], sems[b]).wait()
        plsc.subcore_barrier()
        pltpu.sync_copy(acc.at[pl.ds(s * RPT, RPT)],
                        out_hbm.at[c, h, pl.ds(s * RPT, RPT)])
        plsc.subcore_barrier()


_RB = 1000
_GRID = NN // _RB


def _pre_body(x_ref, w_ref, degp_ref, ulo_ref, uhi_ref, dis_ref):
    deg = degp_ref[0] + degp_ref[1] + 1.0
    dis = lax.rsqrt(deg)
    g = jnp.dot(x_ref[...], w_ref[...],
                preferred_element_type=jnp.float32) * dis
    ulo_ref[...] = g[:, :DH]
    uhi_ref[...] = g[:, DH:]
    dis_ref[...] = dis


def _mid_body(sp_ref, ulo_ref, uhi_ref, dis_ref, b_ref, w_ref,
              olo_ref, ohi_ref):
    tlo = sp_ref[0, 0] + sp_ref[1, 0] + ulo_ref[...]
    thi = sp_ref[0, 1] + sp_ref[1, 1] + uhi_ref[...]
    t = jnp.concatenate([tlo, thi], axis=-1)
    q = jnp.maximum(t * dis_ref[...] + b_ref[...], 0.0)
    g = jnp.dot(q, w_ref[...], preferred_element_type=jnp.float32) * dis_ref[...]
    olo_ref[...] = g[:, :DH]
    ohi_ref[...] = g[:, DH:]


def _post_body(sp_ref, ulo_ref, uhi_ref, dis_ref, b_ref, out_ref):
    tlo = sp_ref[0, 0] + sp_ref[1, 0] + ulo_ref[...]
    thi = sp_ref[0, 1] + sp_ref[1, 1] + uhi_ref[...]
    t = jnp.concatenate([tlo, thi], axis=-1)
    out_ref[...] = t * dis_ref[...] + b_ref[...]


def _tc_pre(x, W1, degp2d):
    return pl.pallas_call(
        _pre_body,
        grid=(_GRID,),
        in_specs=[
            pl.BlockSpec((_RB, DD), lambda i: (i, 0)),
            pl.BlockSpec((DD, DD), lambda i: (0, 0)),
            pl.BlockSpec((2, _RB, 1), lambda i: (0, i, 0)),
        ],
        out_specs=[
            pl.BlockSpec((_RB, DH), lambda i: (i, 0)),
            pl.BlockSpec((_RB, DH), lambda i: (i, 0)),
            pl.BlockSpec((_RB, 1), lambda i: (i, 0)),
        ],
        out_shape=[
            jax.ShapeDtypeStruct((NN, DH), jnp.float32),
            jax.ShapeDtypeStruct((NN, DH), jnp.float32),
            jax.ShapeDtypeStruct((NN, 1), jnp.float32),
        ],
    )(x, W1, degp2d)


def _tc_mid(sp, ulo, uhi, dis, b, W):
    return pl.pallas_call(
        _mid_body,
        grid=(_GRID,),
        in_specs=[
            pl.BlockSpec((2, 2, _RB, DH), lambda i: (0, 0, i, 0)),
            pl.BlockSpec((_RB, DH), lambda i: (i, 0)),
            pl.BlockSpec((_RB, DH), lambda i: (i, 0)),
            pl.BlockSpec((_RB, 1), lambda i: (i, 0)),
            pl.BlockSpec((1, DD), lambda i: (0, 0)),
            pl.BlockSpec((DD, DD), lambda i: (0, 0)),
        ],
        out_specs=[
            pl.BlockSpec((_RB, DH), lambda i: (i, 0)),
            pl.BlockSpec((_RB, DH), lambda i: (i, 0)),
        ],
        out_shape=[
            jax.ShapeDtypeStruct((NN, DH), jnp.float32),
            jax.ShapeDtypeStruct((NN, DH), jnp.float32),
        ],
    )(sp, ulo, uhi, dis, b, W)


def _tc_post(sp, ulo, uhi, dis, b):
    return pl.pallas_call(
        _post_body,
        grid=(_GRID,),
        in_specs=[
            pl.BlockSpec((2, 2, _RB, DH), lambda i: (0, 0, i, 0)),
            pl.BlockSpec((_RB, DH), lambda i: (i, 0)),
            pl.BlockSpec((_RB, DH), lambda i: (i, 0)),
            pl.BlockSpec((_RB, 1), lambda i: (i, 0)),
            pl.BlockSpec((1, DD), lambda i: (0, 0)),
        ],
        out_specs=pl.BlockSpec((_RB, DD), lambda i: (i, 0)),
        out_shape=jax.ShapeDtypeStruct((NN, DD), jnp.float32),
    )(sp, ulo, uhi, dis, b)


def _build_edge_slabs(src, dst, w):
    padlen = NCHUNK * EC - EPT
    srcs = src.reshape(NW, EPT)
    dsts = dst.reshape(NW, EPT)
    ws = w.reshape(NW, EPT)
    pad_src = ((jnp.arange(padlen, dtype=jnp.int32) * 41) % NN)[None, :]
    pad_dst = (NN + jnp.arange(padlen, dtype=jnp.int32) % (NPAD - NN))[None, :]
    pad_w = jnp.zeros((1, padlen), jnp.float32)
    srcs = jnp.concatenate(
        [srcs, jnp.broadcast_to(pad_src, (NW, padlen))], axis=1)
    dsts = jnp.concatenate(
        [dsts, jnp.broadcast_to(pad_dst, (NW, padlen))], axis=1)
    ws = jnp.concatenate([ws, jnp.broadcast_to(pad_w, (NW, padlen))], axis=1)
    d_src = jnp.broadcast_to(
        ((jnp.arange(2 * EC, dtype=jnp.int32) * 79) % NN)[None, :],
        (NW, 2 * EC))
    d_dst = jnp.broadcast_to(
        (NN + jnp.arange(2 * EC, dtype=jnp.int32) % (NPAD - NN))[None, :],
        (NW, 2 * EC))
    srcs_g = jnp.concatenate([srcs, d_src], axis=1)
    dsts_g = jnp.concatenate([dsts, d_dst], axis=1)
    packed = (srcs_g | (dsts_g << 14)).reshape(NW, NGCHUNK, EC)
    w_slab = ws.reshape(NW, NCHUNK, EC)
    return packed, w_slab


def kernel(x, edge_index, edge_attr, W1, b1, W2, b2, W3, b3, W4, b4):
    src = edge_index[0]
    dst = edge_index[1]
    pk_slab, w_slab = _build_edge_slabs(src, dst, edge_attr)

    degp = _deg_kernel(pk_slab, w_slab)
    degp2d = degp[:, :NN, None]

    u1lo, u1hi, dis = _tc_pre(x, W1, degp2d)
    s1 = _spmm_kernel(pk_slab, w_slab, u1lo, u1hi)
    u2lo, u2hi = _tc_mid(s1, u1lo, u1hi, dis, b1.reshape(1, DD), W2)
    s2 = _spmm_kernel(pk_slab, w_slab, u2lo, u2hi)
    u3lo, u3hi = _tc_mid(s2, u2lo, u2hi, dis, b2.reshape(1, DD), W3)
    s3 = _spmm_kernel(pk_slab, w_slab, u3lo, u3hi)
    u4lo, u4hi = _tc_mid(s3, u3lo, u3hi, dis, b3.reshape(1, DD), W4)
    s4 = _spmm_kernel(pk_slab, w_slab, u4lo, u4hi)
    return _tc_post(s4, u4lo, u4hi, dis, b4.reshape(1, DD))


# speedup vs baseline: 18.6808x; 2.4914x over previous
import functools

import jax
import jax.numpy as jnp
from jax import lax
from jax.experimental import pallas as pl
from jax.experimental.pallas import tpu as pltpu
from jax.experimental.pallas import tpu_sc as plsc

NN = 10000
EE = 320000
DD = 128
DH = DD // 2
NC = 2
NS = 16
NW = NC * NS
NPAD = 10240
RPT = NPAD // NS
EC = 128
EPT = EE // NW
NCHUNK = 80
NGCHUNK = NCHUNK + 2

_MESH = plsc.VectorSubcoreMesh(core_axis_name="c", subcore_axis_name="s")
_SC_PARAMS = pltpu.CompilerParams(use_tc_tiling_on_sc=False)


def _unpack_indices(pk_v, dst_v, nchunk):
    def _body(i, carry):
        r = i // 8
        sl = pl.ds((i % 8) * 16, 16)
        v = pk_v[r, sl]
        dst_v[r, sl] = lax.shift_right_logical(v, 14)
        pk_v[r, sl] = v & 0x3FFF
        return carry

    lax.fori_loop(0, nchunk * 8, _body, 0)


@functools.partial(
    pl.kernel,
    out_type=jax.ShapeDtypeStruct((NC, NPAD), jnp.float32),
    mesh=_MESH,
    compiler_params=_SC_PARAMS,
    scratch_types=[
        pltpu.VMEM((NCHUNK, EC), jnp.int32),
        pltpu.VMEM((NCHUNK, EC), jnp.int32),
        pltpu.VMEM((NCHUNK, EC), jnp.float32),
        pltpu.VMEM((RPT,), jnp.float32),
        pltpu.VMEM_SHARED((NPAD,), jnp.float32),
    ],
)
def _deg_kernel(pk_hbm, w_hbm, deg_hbm, pk_v, dst_v, w_v, zbuf, acc):
    c = lax.axis_index("c")
    s = lax.axis_index("s")
    wid = c * NS + s
    pltpu.sync_copy(pk_hbm.at[wid, pl.ds(0, NCHUNK)], pk_v)
    pltpu.sync_copy(w_hbm.at[wid], w_v)
    _unpack_indices(pk_v, dst_v, NCHUNK)

    def _zero(i, carry):
        zbuf[pl.ds(i * 16, 16)] = jnp.zeros((16,), jnp.float32)
        return carry

    lax.fori_loop(0, RPT // 16, _zero, 0)
    pltpu.sync_copy(zbuf, acc.at[pl.ds(s * RPT, RPT)])
    plsc.subcore_barrier()

    def _chunk(j, carry):
        pltpu.sync_copy(w_v.at[j], acc.at[dst_v.at[j]], add=True)
        return carry

    lax.fori_loop(0, NCHUNK, _chunk, 0)
    plsc.subcore_barrier()
    pltpu.sync_copy(acc.at[pl.ds(s * RPT, RPT)],
                    deg_hbm.at[c, pl.ds(s * RPT, RPT)])


@functools.partial(
    pl.kernel,
    out_type=jax.ShapeDtypeStruct((NC, 2, NPAD, DH), jnp.float32),
    mesh=_MESH,
    compiler_params=_SC_PARAMS,
    scratch_types=[
        pltpu.VMEM((NGCHUNK, EC), jnp.int32),
        pltpu.VMEM((NGCHUNK, EC), jnp.int32),
        pltpu.VMEM((NCHUNK, EC), jnp.float32),
        pltpu.VMEM((4, EC, DH), jnp.float32),
        [pltpu.SemaphoreType.DMA] * 4,
        [pltpu.SemaphoreType.DMA] * 4,
        pltpu.VMEM_SHARED((NPAD, DH), jnp.float32),
    ],
)
def _spmm_kernel(pk_hbm, w_hbm, ulo_hbm, uhi_hbm, out_hbm,
                 src_v, dst_v, w_v, rows_v, gsems, ssems, acc):
    c = lax.axis_index("c")
    s = lax.axis_index("s")
    wid = c * NS + s
    pltpu.sync_copy(pk_hbm.at[wid], src_v)
    pltpu.sync_copy(w_hbm.at[wid], w_v)
    _unpack_indices(src_v, dst_v, NGCHUNK)

    for h, u_hbm in enumerate((ulo_hbm, uhi_hbm)):
        def _zero(i, carry):
            for g in range(DH // 16):
                rows_v[0, i, pl.ds(g * 16, 16)] = jnp.zeros((16,), jnp.float32)
            return carry

        lax.fori_loop(0, EC, _zero, 0)
        for k in range(RPT // EC):
            pltpu.sync_copy(rows_v.at[0], acc.at[pl.ds(s * RPT + k * EC, EC)])
        plsc.subcore_barrier()

        def _fire_gather(jj, b):
            pltpu.async_copy(u_hbm.at[src_v.at[jj]], rows_v.at[b], gsems[b])

        def _wait_gather(jj, b):
            pltpu.make_async_copy(u_hbm.at[src_v.at[jj]], rows_v.at[b],
                                  gsems[b]).wait()

        def _fire_scatter(jj, b):
            pltpu.async_copy(rows_v.at[b], acc.at[dst_v.at[jj]], ssems[b],
                             add=True)

        def _wait_scatter(jj, b):
            pltpu.make_async_copy(rows_v.at[b], acc.at[dst_v.at[jj]],
                                  ssems[b]).wait()

        def _scale(jj, b):
            def _grp(g16, carry):
                w16 = w_v[jj, pl.ds(g16 * 16, 16)]
                for e4 in range(4):
                    base = g16 * 16 + e4 * 4
                    wls = [w16[e4 * 4 + i] for i in range(4)]
                    vals = [rows_v[b, base + i, pl.ds(g * 16, 16)]
                            for i in range(4) for g in range(DH // 16)]
                    prods = [v * wls[i // (DH // 16)]
                             for i, v in enumerate(vals)]
                    for i in range(4):
                        for g in range(DH // 16):
                            rows_v[b, base + i, pl.ds(g * 16, 16)] = (
                                prods[i * (DH // 16) + g])
                return carry

            lax.fori_loop(0, EC // 16, _grp, 0, unroll=4)

        for jj in range(2):
            _fire_gather(jj, jj)

        def _slot(jj, b):
            b2 = (b + 2) % 4

            @pl.when(jnp.logical_and(jj >= 2, jj < NCHUNK + 2))
            def _():
                _wait_scatter(jj - 2, b2)

            @pl.when(jj < NCHUNK)
            def _():
                _fire_gather(jj + 2, b2)

            @pl.when(jj < NCHUNK)
            def _():
                _wait_gather(jj, b)
                _scale(jj, b)
                _fire_scatter(jj, b)

            @pl.when(jnp.logical_and(jj >= NCHUNK, jj < NCHUNK + 2))
            def _():
                _wait_gather(jj, b)

        def _outer(i, carry):
            for bo in range(4):
                _slot(i * 4 + bo, bo)
            return carry

        lax.fori_loop(0, (NCHUNK + 4) // 4, _outer, 0)
        plsc.subcore_barrier()
        pltpu.sync_copy(acc.at[pl.ds(s * RPT, RPT)],
                        out_hbm.at[c, h, pl.ds(s * RPT, RPT)])
        plsc.subcore_barrier()


_RB = 1000
_GRID = NN // _RB


def _pre_body(x_ref, w_ref, degp_ref, ulo_ref, uhi_ref, dis_ref):
    deg = degp_ref[0] + degp_ref[1] + 1.0
    dis = lax.rsqrt(deg)
    g = jnp.dot(x_ref[...], w_ref[...],
                preferred_element_type=jnp.float32) * dis
    ulo_ref[...] = g[:, :DH]
    uhi_ref[...] = g[:, DH:]
    dis_ref[...] = dis


def _mid_body(sp_ref, ulo_ref, uhi_ref, dis_ref, b_ref, w_ref,
              olo_ref, ohi_ref):
    tlo = sp_ref[0, 0] + sp_ref[1, 0] + ulo_ref[...]
    thi = sp_ref[0, 1] + sp_ref[1, 1] + uhi_ref[...]
    t = jnp.concatenate([tlo, thi], axis=-1)
    q = jnp.maximum(t * dis_ref[...] + b_ref[...], 0.0)
    g = jnp.dot(q, w_ref[...], preferred_element_type=jnp.float32) * dis_ref[...]
    olo_ref[...] = g[:, :DH]
    ohi_ref[...] = g[:, DH:]


def _post_body(sp_ref, ulo_ref, uhi_ref, dis_ref, b_ref, out_ref):
    tlo = sp_ref[0, 0] + sp_ref[1, 0] + ulo_ref[...]
    thi = sp_ref[0, 1] + sp_ref[1, 1] + uhi_ref[...]
    t = jnp.concatenate([tlo, thi], axis=-1)
    out_ref[...] = t * dis_ref[...] + b_ref[...]


def _tc_pre(x, W1, degp2d):
    return pl.pallas_call(
        _pre_body,
        grid=(_GRID,),
        in_specs=[
            pl.BlockSpec((_RB, DD), lambda i: (i, 0)),
            pl.BlockSpec((DD, DD), lambda i: (0, 0)),
            pl.BlockSpec((2, _RB, 1), lambda i: (0, i, 0)),
        ],
        out_specs=[
            pl.BlockSpec((_RB, DH), lambda i: (i, 0)),
            pl.BlockSpec((_RB, DH), lambda i: (i, 0)),
            pl.BlockSpec((_RB, 1), lambda i: (i, 0)),
        ],
        out_shape=[
            jax.ShapeDtypeStruct((NN, DH), jnp.float32),
            jax.ShapeDtypeStruct((NN, DH), jnp.float32),
            jax.ShapeDtypeStruct((NN, 1), jnp.float32),
        ],
    )(x, W1, degp2d)


def _tc_mid(sp, ulo, uhi, dis, b, W):
    return pl.pallas_call(
        _mid_body,
        grid=(_GRID,),
        in_specs=[
            pl.BlockSpec((2, 2, _RB, DH), lambda i: (0, 0, i, 0)),
            pl.BlockSpec((_RB, DH), lambda i: (i, 0)),
            pl.BlockSpec((_RB, DH), lambda i: (i, 0)),
            pl.BlockSpec((_RB, 1), lambda i: (i, 0)),
            pl.BlockSpec((1, DD), lambda i: (0, 0)),
            pl.BlockSpec((DD, DD), lambda i: (0, 0)),
        ],
        out_specs=[
            pl.BlockSpec((_RB, DH), lambda i: (i, 0)),
            pl.BlockSpec((_RB, DH), lambda i: (i, 0)),
        ],
        out_shape=[
            jax.ShapeDtypeStruct((NN, DH), jnp.float32),
            jax.ShapeDtypeStruct((NN, DH), jnp.float32),
        ],
    )(sp, ulo, uhi, dis, b, W)


def _tc_post(sp, ulo, uhi, dis, b):
    return pl.pallas_call(
        _post_body,
        grid=(_GRID,),
        in_specs=[
            pl.BlockSpec((2, 2, _RB, DH), lambda i: (0, 0, i, 0)),
            pl.BlockSpec((_RB, DH), lambda i: (i, 0)),
            pl.BlockSpec((_RB, DH), lambda i: (i, 0)),
            pl.BlockSpec((_RB, 1), lambda i: (i, 0)),
            pl.BlockSpec((1, DD), lambda i: (0, 0)),
        ],
        out_specs=pl.BlockSpec((_RB, DD), lambda i: (i, 0)),
        out_shape=jax.ShapeDtypeStruct((NN, DD), jnp.float32),
    )(sp, ulo, uhi, dis, b)


def _build_edge_slabs(src, dst, w):
    padlen = NCHUNK * EC - EPT
    srcs = src.reshape(NW, EPT)
    dsts = dst.reshape(NW, EPT)
    ws = w.reshape(NW, EPT)
    pad_src = ((jnp.arange(padlen, dtype=jnp.int32) * 41) % NN)[None, :]
    pad_dst = (NN + jnp.arange(padlen, dtype=jnp.int32) % (NPAD - NN))[None, :]
    pad_w = jnp.zeros((1, padlen), jnp.float32)
    srcs = jnp.concatenate(
        [srcs, jnp.broadcast_to(pad_src, (NW, padlen))], axis=1)
    dsts = jnp.concatenate(
        [dsts, jnp.broadcast_to(pad_dst, (NW, padlen))], axis=1)
    ws = jnp.concatenate([ws, jnp.broadcast_to(pad_w, (NW, padlen))], axis=1)
    d_src = jnp.broadcast_to(
        ((jnp.arange(2 * EC, dtype=jnp.int32) * 79) % NN)[None, :],
        (NW, 2 * EC))
    d_dst = jnp.broadcast_to(
        (NN + jnp.arange(2 * EC, dtype=jnp.int32) % (NPAD - NN))[None, :],
        (NW, 2 * EC))
    srcs_g = jnp.concatenate([srcs, d_src], axis=1)
    dsts_g = jnp.concatenate([dsts, d_dst], axis=1)
    packed = (srcs_g | (dsts_g << 14)).reshape(NW, NGCHUNK, EC)
    w_slab = ws.reshape(NW, NCHUNK, EC)
    return packed, w_slab


def kernel(x, edge_index, edge_attr, W1, b1, W2, b2, W3, b3, W4, b4):
    src = edge_index[0]
    dst = edge_index[1]
    pk_slab, w_slab = _build_edge_slabs(src, dst, edge_attr)

    degp = _deg_kernel(pk_slab, w_slab)
    degp2d = degp[:, :NN, None]

    u1lo, u1hi, dis = _tc_pre(x, W1, degp2d)
    s1 = _spmm_kernel(pk_slab, w_slab, u1lo, u1hi)
    u2lo, u2hi = _tc_mid(s1, u1lo, u1hi, dis, b1.reshape(1, DD), W2)
    s2 = _spmm_kernel(pk_slab, w_slab, u2lo, u2hi)
    u3lo, u3hi = _tc_mid(s2, u2lo, u2hi, dis, b2.reshape(1, DD), W3)
    s3 = _spmm_kernel(pk_slab, w_slab, u3lo, u3hi)
    u4lo, u4hi = _tc_mid(s3, u3lo, u3hi, dis, b3.reshape(1, DD), W4)
    s4 = _spmm_kernel(pk_slab, w_slab, u4lo, u4hi)
    return _tc_post(s4, u4lo, u4hi, dis, b4.reshape(1, DD))
